# baseline (speedup 1.0000x reference)
import functools

import jax
import jax.numpy as jnp
from jax import lax
from jax.experimental import pallas as pl
from jax.experimental.pallas import tpu as pltpu
from jax.experimental.pallas import tpu_sc as plsc

NV = 50000
NC = 100000
CL = 8
B = 16

BCOLS = 784
BITER = 4
NW = 32
CCH = 160
LCH = CCH * CL
NGRP = 10
GSZ = LCH // NGRP
NCH = NC // CCH
KPW = 20
TG = CCH // 16


def _sc_body(s_hbm, var_hbm, neg_hbm, out_hbm, t_hbm,
             wbuf, tbuf, vn_v, idx_v, rows_v, obuf,
             sem_v0, sem_v1, sem_g0, sem_g1):
    cid = lax.axis_index("c")
    sid = lax.axis_index("s")
    w = sid * 2 + cid
    iota16 = lax.iota(jnp.int32, 16)
    sem_v = (sem_v0, sem_v1)
    sem_g = (sem_g0, sem_g1)

    def drain_build():
        pltpu.make_async_copy(
            tbuf.at[0, 0], t_hbm.at[0, pl.ds(0, BCOLS)], sem_g0).wait()
        pltpu.make_async_copy(
            tbuf.at[0, 0], t_hbm.at[0, pl.ds(0, BCOLS)], sem_g0).wait()

    for it in range(BITER):
        r = sid * BITER + it
        start = jnp.minimum(r * BCOLS, NV - BCOLS)
        pltpu.sync_copy(s_hbm.at[:, pl.ds(start, BCOLS)], wbuf)
        if it >= 2:
            drain_build()
        tb = tbuf.at[it % 2]

        def build_col(u, z):
            x = plsc.load_gather(
                wbuf, [iota16, jnp.full((16,), 0, jnp.int32) + u])
            tb[0, u, :] = 1.0 - x
            tb[1, u, :] = x
            return z

        lax.fori_loop(0, BCOLS, build_col, 0, unroll=4)
        pltpu.async_copy(
            tb.at[0], t_hbm.at[cid, pl.ds(start, BCOLS)], sem_g0)
        pltpu.async_copy(
            tb.at[1], t_hbm.at[cid, pl.ds(NV + start, BCOLS)], sem_g0)
    drain_build()
    drain_build()
    plsc.subcore_barrier()

    c0 = w * KPW

    def stage_vn_async(p, chunk):
        lit = jnp.minimum(chunk, NCH - 1) * LCH
        pltpu.async_copy(var_hbm.at[pl.ds(lit, LCH)], vn_v.at[p, 0], sem_v[p])
        pltpu.async_copy(neg_hbm.at[pl.ds(lit, LCH)], vn_v.at[p, 1], sem_v[p])

    def wait_vn(p):
        pltpu.make_async_copy(
            var_hbm.at[pl.ds(0, LCH)], vn_v.at[0, 0], sem_v[p]).wait()
        pltpu.make_async_copy(
            neg_hbm.at[pl.ds(0, LCH)], vn_v.at[0, 1], sem_v[p]).wait()

    def cidx_compute(p):
        def body(i, z):
            v = vn_v[p, 0, pl.ds(i * 16, 16)]
            n = vn_v[p, 1, pl.ds(i * 16, 16)]
            idx_v[p, pl.ds(i * 16, 16)] = n * NV + v - 1
            return z
        lax.fori_loop(0, LCH // 16, body, 0, unroll=8)

    def fire_gathers(p):
        for j in range(NGRP):
            pltpu.async_copy(
                t_hbm.at[cid].at[idx_v.at[p, pl.ds(j * GSZ, GSZ)]],
                rows_v.at[p, pl.ds(j * GSZ, GSZ)],
                sem_g[p],
            )

    def wait_gathers(p):
        pltpu.make_async_copy(
            t_hbm.at[0, pl.ds(0, LCH)], rows_v.at[0], sem_g[p]).wait()

    def products(p):
        def body(ci, z):
            base = ci * CL
            r = [rows_v[p, base + l, :] for l in range(CL)]
            acc = ((r[0] * r[1]) * (r[2] * r[3])) * \
                  ((r[4] * r[5]) * (r[6] * r[7]))
            plsc.store_scatter(
                obuf, [iota16, jnp.full((16,), 0, jnp.int32) + ci],
                1.0 - acc)
            return z
        lax.fori_loop(0, CCH, body, 0, unroll=8)

    def write_out(chunk):
        ce = jnp.minimum(chunk, NCH - 1)
        pltpu.sync_copy(obuf, out_hbm.at[:, pl.ds(ce * CCH, CCH)])

    lit0 = jnp.minimum(c0, NCH - 1) * LCH
    pltpu.sync_copy(var_hbm.at[pl.ds(lit0, LCH)], vn_v.at[0, 0])
    pltpu.sync_copy(neg_hbm.at[pl.ds(lit0, LCH)], vn_v.at[0, 1])
    cidx_compute(0)
    fire_gathers(0)
    stage_vn_async(1, c0 + 1)

    def pair_body(t, z):
        k = c0 + 2 * t
        wait_gathers(0)
        wait_vn(1)
        cidx_compute(1)
        fire_gathers(1)
        stage_vn_async(0, k + 2)
        products(0)
        write_out(k)
        wait_gathers(1)
        wait_vn(0)
        cidx_compute(0)
        fire_gathers(0)
        stage_vn_async(1, k + 3)
        products(1)
        write_out(k + 1)
        return z

    lax.fori_loop(0, KPW // 2 - 1, pair_body, 0)

    wait_gathers(0)
    wait_vn(1)
    cidx_compute(1)
    fire_gathers(1)
    products(0)
    write_out(c0 + KPW - 2)
    wait_gathers(1)
    products(1)
    write_out(c0 + KPW - 1)


@jax.jit
def _run(s, var_idx, neg):
    mesh = plsc.VectorSubcoreMesh(core_axis_name="c", subcore_axis_name="s")
    sc = functools.partial(
        pl.kernel,
        mesh=mesh,
        out_type=(
            jax.ShapeDtypeStruct((B, NC), jnp.float32),
            jax.ShapeDtypeStruct((2, 2 * NV, 16), jnp.float32),
        ),
        scratch_types=[
            pltpu.VMEM((16, BCOLS), jnp.float32),
            pltpu.VMEM((2, 2, BCOLS, 16), jnp.float32),
            pltpu.VMEM((2, 2, LCH), jnp.int32),
            pltpu.VMEM((2, LCH), jnp.int32),
            pltpu.VMEM((2, LCH, 16), jnp.float32),
            pltpu.VMEM((16, CCH), jnp.float32),
            pltpu.SemaphoreType.DMA,
            pltpu.SemaphoreType.DMA,
            pltpu.SemaphoreType.DMA,
            pltpu.SemaphoreType.DMA,
        ],
        compiler_params=pltpu.CompilerParams(
            use_tc_tiling_on_sc=False, needs_layout_passes=False,
            disable_bounds_checks=True),
    )(_sc_body)
    return sc(s, var_idx, neg)


def kernel(input, emb_weight, var_idx, neg):
    w = jnp.take(emb_weight, input, axis=0)
    s = jax.nn.sigmoid(w)
    out, _ = _run(s, var_idx, neg)
    return out


# speedup vs baseline: 1.0433x; 1.0433x over previous
import functools

import jax
import jax.numpy as jnp
from jax import lax
from jax.experimental import pallas as pl
from jax.experimental.pallas import tpu as pltpu
from jax.experimental.pallas import tpu_sc as plsc

NV = 50000
NC = 100000
CL = 8
B = 16

BCOLS = 784
BITER = 4
NW = 32
CCH = 160
LCH = CCH * CL
NGRP = 10
GSZ = LCH // NGRP
NCH = NC // CCH
KPW = 20
TG = CCH // 16


def _sc_body(s_hbm, cidx_hbm, out_hbm, t_hbm,
             wbuf, tbuf, idx_v, rows_v, obuf,
             sem_v0, sem_v1, sem_g0, sem_g1):
    cid = lax.axis_index("c")
    sid = lax.axis_index("s")
    w = sid * 2 + cid
    iota16 = lax.iota(jnp.int32, 16)
    sem_v = (sem_v0, sem_v1)
    sem_g = (sem_g0, sem_g1)

    def drain_build():
        pltpu.make_async_copy(
            tbuf.at[0, 0], t_hbm.at[0, pl.ds(0, BCOLS)], sem_g0).wait()
        pltpu.make_async_copy(
            tbuf.at[0, 0], t_hbm.at[0, pl.ds(0, BCOLS)], sem_g0).wait()

    for it in range(BITER):
        r = sid * BITER + it
        start = jnp.minimum(r * BCOLS, NV - BCOLS)
        pltpu.sync_copy(s_hbm.at[:, pl.ds(start, BCOLS)], wbuf)
        if it >= 2:
            drain_build()
        tb = tbuf.at[it % 2]

        def build_col(u, z):
            x = plsc.load_gather(
                wbuf, [iota16, jnp.full((16,), 0, jnp.int32) + u])
            tb[0, u, :] = 1.0 - x
            tb[1, u, :] = x
            return z

        lax.fori_loop(0, BCOLS, build_col, 0, unroll=4)
        pltpu.async_copy(
            tb.at[0], t_hbm.at[cid, pl.ds(start, BCOLS)], sem_g0)
        pltpu.async_copy(
            tb.at[1], t_hbm.at[cid, pl.ds(NV + start, BCOLS)], sem_g0)
    drain_build()
    drain_build()
    plsc.subcore_barrier()

    c0 = w * KPW

    def stage_vn_async(p, chunk):
        lit = jnp.minimum(chunk, NCH - 1) * LCH
        pltpu.async_copy(cidx_hbm.at[pl.ds(lit, LCH)], idx_v.at[p], sem_v[p])

    def wait_vn(p):
        pltpu.make_async_copy(
            cidx_hbm.at[pl.ds(0, LCH)], idx_v.at[0], sem_v[p]).wait()

    def fire_gathers(p):
        for j in range(NGRP):
            pltpu.async_copy(
                t_hbm.at[cid].at[idx_v.at[p, pl.ds(j * GSZ, GSZ)]],
                rows_v.at[p, pl.ds(j * GSZ, GSZ)],
                sem_g[p],
            )

    def wait_gathers(p):
        pltpu.make_async_copy(
            t_hbm.at[0, pl.ds(0, LCH)], rows_v.at[0], sem_g[p]).wait()

    def products(p):
        def body(ci, z):
            base = ci * CL
            r = [rows_v[p, base + l, :] for l in range(CL)]
            acc = ((r[0] * r[1]) * (r[2] * r[3])) * \
                  ((r[4] * r[5]) * (r[6] * r[7]))
            plsc.store_scatter(
                obuf, [iota16, jnp.full((16,), 0, jnp.int32) + ci],
                1.0 - acc)
            return z
        lax.fori_loop(0, CCH, body, 0, unroll=8)

    def write_out(chunk):
        ce = jnp.minimum(chunk, NCH - 1)
        pltpu.sync_copy(obuf, out_hbm.at[:, pl.ds(ce * CCH, CCH)])

    lit0 = jnp.minimum(c0, NCH - 1) * LCH
    pltpu.sync_copy(cidx_hbm.at[pl.ds(lit0, LCH)], idx_v.at[0])
    fire_gathers(0)
    stage_vn_async(1, c0 + 1)

    def pair_body(t, z):
        k = c0 + 2 * t
        wait_gathers(0)
        wait_vn(1)
        fire_gathers(1)
        stage_vn_async(0, k + 2)
        products(0)
        write_out(k)
        wait_gathers(1)
        wait_vn(0)
        fire_gathers(0)
        stage_vn_async(1, k + 3)
        products(1)
        write_out(k + 1)
        return z

    lax.fori_loop(0, KPW // 2 - 1, pair_body, 0)

    wait_gathers(0)
    wait_vn(1)
    fire_gathers(1)
    products(0)
    write_out(c0 + KPW - 2)
    wait_gathers(1)
    products(1)
    write_out(c0 + KPW - 1)


@jax.jit
def _run(s, cidx):
    mesh = plsc.VectorSubcoreMesh(core_axis_name="c", subcore_axis_name="s")
    sc = functools.partial(
        pl.kernel,
        mesh=mesh,
        out_type=(
            jax.ShapeDtypeStruct((B, NC), jnp.float32),
            jax.ShapeDtypeStruct((2, 2 * NV, 16), jnp.float32),
        ),
        scratch_types=[
            pltpu.VMEM((16, BCOLS), jnp.float32),
            pltpu.VMEM((2, 2, BCOLS, 16), jnp.float32),
            pltpu.VMEM((2, LCH), jnp.int32),
            pltpu.VMEM((2, LCH, 16), jnp.float32),
            pltpu.VMEM((16, CCH), jnp.float32),
            pltpu.SemaphoreType.DMA,
            pltpu.SemaphoreType.DMA,
            pltpu.SemaphoreType.DMA,
            pltpu.SemaphoreType.DMA,
        ],
        compiler_params=pltpu.CompilerParams(
            use_tc_tiling_on_sc=False, needs_layout_passes=False,
            disable_bounds_checks=True),
    )(_sc_body)
    return sc(s, cidx)


def kernel(input, emb_weight, var_idx, neg):
    w = jnp.take(emb_weight, input, axis=0, mode='clip')
    s = jax.nn.sigmoid(w)
    cidx = neg * NV + var_idx - 1
    out, _ = _run(s, cidx)
    return out


# speedup vs baseline: 1.1377x; 1.0905x over previous
import functools

import jax
import jax.numpy as jnp
from jax import lax
from jax.experimental import pallas as pl
from jax.experimental.pallas import tpu as pltpu
from jax.experimental.pallas import tpu_sc as plsc

NV = 50000
NC = 100000
CL = 8
B = 16

BCOLS = 784
BITER = 4
NW = 32
CCH = 160
LCH = CCH * CL
NGRP = 10
GSZ = LCH // NGRP
NCH = NC // CCH
KPW = 20
TG = CCH // 16


def _sc_body(s_hbm, cidx_hbm, out_hbm, t_hbm,
             wbuf, tbuf, idx_v, rows_v, obuf,
             sem_v0, sem_v1, sem_g0, sem_g1):
    cid = lax.axis_index("c")
    sid = lax.axis_index("s")
    w = sid * 2 + cid
    iota16 = lax.iota(jnp.int32, 16)
    sem_v = (sem_v0, sem_v1)
    sem_g = (sem_g0, sem_g1)

    def drain_build():
        pltpu.make_async_copy(
            tbuf.at[0, 0], t_hbm.at[0, pl.ds(0, BCOLS)], sem_g0).wait()
        pltpu.make_async_copy(
            tbuf.at[0, 0], t_hbm.at[0, pl.ds(0, BCOLS)], sem_g0).wait()

    for it in range(BITER):
        r = sid * BITER + it
        start = jnp.minimum(r * BCOLS, NV - BCOLS)
        pltpu.sync_copy(s_hbm.at[:, pl.ds(start, BCOLS)], wbuf)
        if it >= 2:
            drain_build()
        tb = tbuf.at[it % 2]

        def build_col(u, z):
            x = plsc.load_gather(
                wbuf, [iota16, jnp.full((16,), 0, jnp.int32) + u])
            tb[0, u, :] = 1.0 - x
            tb[1, u, :] = x
            return z

        lax.fori_loop(0, BCOLS, build_col, 0, unroll=4)
        pltpu.async_copy(
            tb.at[0], t_hbm.at[cid, pl.ds(start, BCOLS)], sem_g0)
        pltpu.async_copy(
            tb.at[1], t_hbm.at[cid, pl.ds(NV + start, BCOLS)], sem_g0)
    drain_build()
    drain_build()
    plsc.subcore_barrier()

    c0 = w * KPW

    def stage_vn_async(p, chunk):
        lit = jnp.minimum(chunk, NCH - 1) * LCH
        pltpu.async_copy(cidx_hbm.at[pl.ds(lit, LCH)], idx_v.at[p], sem_v[p])

    def wait_vn(p):
        pltpu.make_async_copy(
            cidx_hbm.at[pl.ds(0, LCH)], idx_v.at[0], sem_v[p]).wait()

    def fire_gathers(p):
        for j in range(NGRP):
            pltpu.async_copy(
                t_hbm.at[cid].at[idx_v.at[p, pl.ds(j * GSZ, GSZ)]],
                rows_v.at[p, pl.ds(j * GSZ, GSZ)],
                sem_g[p],
            )

    def wait_gathers(p):
        pltpu.make_async_copy(
            t_hbm.at[0, pl.ds(0, LCH)], rows_v.at[0], sem_g[p]).wait()

    def products(p):
        def body(ci2, colv):
            base = ci2 * (2 * CL)
            r = [rows_v[p, base + l, :] for l in range(2 * CL)]
            acc0 = ((r[0] * r[1]) * (r[2] * r[3])) * \
                   ((r[4] * r[5]) * (r[6] * r[7]))
            acc1 = ((r[8] * r[9]) * (r[10] * r[11])) * \
                   ((r[12] * r[13]) * (r[14] * r[15]))
            plsc.store_scatter(obuf, [iota16, colv], 1.0 - acc0)
            colv1 = colv + 1
            plsc.store_scatter(obuf, [iota16, colv1], 1.0 - acc1)
            return colv1 + 1
        lax.fori_loop(0, CCH // 2, body,
                      jnp.zeros((16,), jnp.int32), unroll=4)

    def write_out(chunk):
        ce = jnp.minimum(chunk, NCH - 1)
        pltpu.sync_copy(obuf, out_hbm.at[:, pl.ds(ce * CCH, CCH)])

    lit0 = jnp.minimum(c0, NCH - 1) * LCH
    pltpu.sync_copy(cidx_hbm.at[pl.ds(lit0, LCH)], idx_v.at[0])
    fire_gathers(0)
    stage_vn_async(1, c0 + 1)

    def pair_body(t, z):
        k = c0 + 2 * t
        wait_gathers(0)
        wait_vn(1)
        fire_gathers(1)
        stage_vn_async(0, k + 2)
        products(0)
        write_out(k)
        wait_gathers(1)
        wait_vn(0)
        fire_gathers(0)
        stage_vn_async(1, k + 3)
        products(1)
        write_out(k + 1)
        return z

    lax.fori_loop(0, KPW // 2 - 1, pair_body, 0)

    wait_gathers(0)
    wait_vn(1)
    fire_gathers(1)
    products(0)
    write_out(c0 + KPW - 2)
    wait_gathers(1)
    products(1)
    write_out(c0 + KPW - 1)


@jax.jit
def _run(s, cidx):
    mesh = plsc.VectorSubcoreMesh(core_axis_name="c", subcore_axis_name="s")
    sc = functools.partial(
        pl.kernel,
        mesh=mesh,
        out_type=(
            jax.ShapeDtypeStruct((B, NC), jnp.float32),
            jax.ShapeDtypeStruct((2, 2 * NV, 16), jnp.float32),
        ),
        scratch_types=[
            pltpu.VMEM((16, BCOLS), jnp.float32),
            pltpu.VMEM((2, 2, BCOLS, 16), jnp.float32),
            pltpu.VMEM((2, LCH), jnp.int32),
            pltpu.VMEM((2, LCH, 16), jnp.float32),
            pltpu.VMEM((16, CCH), jnp.float32),
            pltpu.SemaphoreType.DMA,
            pltpu.SemaphoreType.DMA,
            pltpu.SemaphoreType.DMA,
            pltpu.SemaphoreType.DMA,
        ],
        compiler_params=pltpu.CompilerParams(
            use_tc_tiling_on_sc=False, needs_layout_passes=False,
            disable_bounds_checks=True),
    )(_sc_body)
    return sc(s, cidx)


def kernel(input, emb_weight, var_idx, neg):
    w = jnp.take(emb_weight, input, axis=0, mode='clip')
    s = jax.nn.sigmoid(w)
    cidx = neg * NV + var_idx - 1
    out, _ = _run(s, cidx)
    return out


# speedup vs baseline: 1.2350x; 1.0856x over previous
import functools

import jax
import jax.numpy as jnp
from jax import lax
from jax.experimental import pallas as pl
from jax.experimental.pallas import tpu as pltpu
from jax.experimental.pallas import tpu_sc as plsc

NV = 50000
NC = 100000
CL = 8
B = 16

BCOLS = 784
BITER = 4
NW = 32
CCH = 160
LCH = CCH * CL
NGRP = 10
GSZ = LCH // NGRP
NCH = NC // CCH
KPW = 20
TG = CCH // 16


def _sc_body(s_hbm, cidx_hbm, out_hbm, t_hbm,
             wbuf, tbuf, idx_v, rows_v, obuf,
             sem_v0, sem_v1, sem_g0, sem_g1):
    cid = lax.axis_index("c")
    sid = lax.axis_index("s")
    w = sid * 2 + cid
    iota16 = lax.iota(jnp.int32, 16)
    sem_v = (sem_v0, sem_v1)
    sem_g = (sem_g0, sem_g1)

    def drain_build():
        pltpu.make_async_copy(
            tbuf.at[0, 0], t_hbm.at[0, pl.ds(0, BCOLS)], sem_g0).wait()
        pltpu.make_async_copy(
            tbuf.at[0, 0], t_hbm.at[0, pl.ds(0, BCOLS)], sem_g0).wait()

    for it in range(BITER):
        r = sid * BITER + it
        start = jnp.minimum(r * BCOLS, NV - BCOLS)
        pltpu.sync_copy(s_hbm.at[:, pl.ds(start, BCOLS)], wbuf)
        if it >= 2:
            drain_build()
        tb = tbuf.at[it % 2]

        def build_pair(u2, colv):
            u = u2 * 2
            x0 = plsc.load_gather(wbuf, [iota16, colv])
            colv1 = colv + 1
            x1 = plsc.load_gather(wbuf, [iota16, colv1])
            tb[0, u, :] = 1.0 - x0
            tb[1, u, :] = x0
            tb[0, u + 1, :] = 1.0 - x1
            tb[1, u + 1, :] = x1
            return colv1 + 1

        lax.fori_loop(0, BCOLS // 2, build_pair,
                      jnp.zeros((16,), jnp.int32), unroll=4)
        pltpu.async_copy(
            tb.at[0], t_hbm.at[cid, pl.ds(start, BCOLS)], sem_g0)
        pltpu.async_copy(
            tb.at[1], t_hbm.at[cid, pl.ds(NV + start, BCOLS)], sem_g0)
    drain_build()
    drain_build()
    plsc.subcore_barrier()

    c0 = w * KPW

    def stage_vn_async(p, chunk):
        lit = jnp.minimum(chunk, NCH - 1) * LCH
        pltpu.async_copy(cidx_hbm.at[pl.ds(lit, LCH)], idx_v.at[p], sem_v[p])

    def wait_vn(p):
        pltpu.make_async_copy(
            cidx_hbm.at[pl.ds(0, LCH)], idx_v.at[0], sem_v[p]).wait()

    def fire_gathers(p):
        for j in range(NGRP):
            pltpu.async_copy(
                t_hbm.at[cid].at[idx_v.at[p, pl.ds(j * GSZ, GSZ)]],
                rows_v.at[p, pl.ds(j * GSZ, GSZ)],
                sem_g[p],
            )

    def wait_gathers(p):
        pltpu.make_async_copy(
            t_hbm.at[0, pl.ds(0, LCH)], rows_v.at[0], sem_g[p]).wait()

    def products(p):
        def body(ci2, colv):
            base = ci2 * (2 * CL)
            r = [rows_v[p, base + l, :] for l in range(2 * CL)]
            acc0 = ((r[0] * r[1]) * (r[2] * r[3])) * \
                   ((r[4] * r[5]) * (r[6] * r[7]))
            acc1 = ((r[8] * r[9]) * (r[10] * r[11])) * \
                   ((r[12] * r[13]) * (r[14] * r[15]))
            plsc.store_scatter(obuf, [iota16, colv], 1.0 - acc0)
            colv1 = colv + 1
            plsc.store_scatter(obuf, [iota16, colv1], 1.0 - acc1)
            return colv1 + 1
        lax.fori_loop(0, CCH // 2, body,
                      jnp.zeros((16,), jnp.int32), unroll=4)

    def write_out(chunk):
        ce = jnp.minimum(chunk, NCH - 1)
        pltpu.sync_copy(obuf, out_hbm.at[:, pl.ds(ce * CCH, CCH)])

    lit0 = jnp.minimum(c0, NCH - 1) * LCH
    pltpu.sync_copy(cidx_hbm.at[pl.ds(lit0, LCH)], idx_v.at[0])
    fire_gathers(0)
    stage_vn_async(1, c0 + 1)

    def pair_body(t, z):
        k = c0 + 2 * t
        wait_gathers(0)
        wait_vn(1)
        fire_gathers(1)
        stage_vn_async(0, k + 2)
        products(0)
        write_out(k)
        wait_gathers(1)
        wait_vn(0)
        fire_gathers(0)
        stage_vn_async(1, k + 3)
        products(1)
        write_out(k + 1)
        return z

    lax.fori_loop(0, KPW // 2 - 1, pair_body, 0)

    wait_gathers(0)
    wait_vn(1)
    fire_gathers(1)
    products(0)
    write_out(c0 + KPW - 2)
    wait_gathers(1)
    products(1)
    write_out(c0 + KPW - 1)


@jax.jit
def _run(s, cidx):
    mesh = plsc.VectorSubcoreMesh(core_axis_name="c", subcore_axis_name="s")
    sc = functools.partial(
        pl.kernel,
        mesh=mesh,
        out_type=(
            jax.ShapeDtypeStruct((B, NC), jnp.float32),
            jax.ShapeDtypeStruct((2, 2 * NV, 16), jnp.float32),
        ),
        scratch_types=[
            pltpu.VMEM((16, BCOLS), jnp.float32),
            pltpu.VMEM((2, 2, BCOLS, 16), jnp.float32),
            pltpu.VMEM((2, LCH), jnp.int32),
            pltpu.VMEM((2, LCH, 16), jnp.float32),
            pltpu.VMEM((16, CCH), jnp.float32),
            pltpu.SemaphoreType.DMA,
            pltpu.SemaphoreType.DMA,
            pltpu.SemaphoreType.DMA,
            pltpu.SemaphoreType.DMA,
        ],
        compiler_params=pltpu.CompilerParams(
            use_tc_tiling_on_sc=False, needs_layout_passes=False,
            disable_bounds_checks=True),
    )(_sc_body)
    return sc(s, cidx)


def kernel(input, emb_weight, var_idx, neg):
    w = jnp.take(emb_weight, input, axis=0, mode='clip')
    s = jax.nn.sigmoid(w)
    cidx = neg * NV + var_idx - 1
    out, _ = _run(s, cidx)
    return out


# speedup vs baseline: 1.2869x; 1.0420x over previous
import functools

import jax
import jax.numpy as jnp
from jax import lax
from jax.experimental import pallas as pl
from jax.experimental.pallas import tpu as pltpu
from jax.experimental.pallas import tpu_sc as plsc

NV = 50000
NC = 100000
CL = 8
B = 16

BCOLS = 784
BITER = 4
NW = 32
CCH = 160
LCH = CCH * CL
NGRP = 10
GSZ = LCH // NGRP
NCH = NC // CCH
KPW = 20
TG = CCH // 16


def _sc_body(s_hbm, cidx_hbm, out_hbm, t_hbm,
             wbuf, tbuf, idx_v, rows_v, obuf,
             sem_v0, sem_v1, sem_g0, sem_g1):
    cid = lax.axis_index("c")
    sid = lax.axis_index("s")
    w = sid * 2 + cid
    iota16 = lax.iota(jnp.int32, 16)
    sem_v = (sem_v0, sem_v1)
    sem_g = (sem_g0, sem_g1)

    def drain_build():
        pltpu.make_async_copy(
            tbuf.at[0, 0], t_hbm.at[0, pl.ds(0, BCOLS)], sem_g0).wait()
        pltpu.make_async_copy(
            tbuf.at[0, 0], t_hbm.at[0, pl.ds(0, BCOLS)], sem_g0).wait()

    for it in range(BITER):
        r = sid * BITER + it
        start = jnp.minimum(r * BCOLS, NV - BCOLS)
        pltpu.sync_copy(s_hbm.at[:, pl.ds(start, BCOLS)], wbuf)
        if it >= 2:
            drain_build()
        tb = tbuf.at[it % 2]

        def build_pair(u2, colv):
            u = u2 * 2
            x0 = plsc.load_gather(wbuf, [iota16, colv])
            colv1 = colv + 1
            x1 = plsc.load_gather(wbuf, [iota16, colv1])
            tb[0, u, :] = 1.0 - x0
            tb[1, u, :] = x0
            tb[0, u + 1, :] = 1.0 - x1
            tb[1, u + 1, :] = x1
            return colv1 + 1

        lax.fori_loop(0, BCOLS // 2, build_pair,
                      jnp.zeros((16,), jnp.int32), unroll=4)
        pltpu.async_copy(
            tb.at[0], t_hbm.at[cid, pl.ds(start, BCOLS)], sem_g0)
        pltpu.async_copy(
            tb.at[1], t_hbm.at[cid, pl.ds(NV + start, BCOLS)], sem_g0)
    drain_build()
    drain_build()
    plsc.subcore_barrier()

    c0 = w * KPW

    def stage_vn_async(p, chunk):
        lit = jnp.minimum(chunk, NCH - 1) * LCH
        pltpu.async_copy(cidx_hbm.at[pl.ds(lit, LCH)], idx_v.at[p], sem_v[p])

    def wait_vn(p):
        pltpu.make_async_copy(
            cidx_hbm.at[pl.ds(0, LCH)], idx_v.at[0], sem_v[p]).wait()

    def fire_gathers(p):
        for j in range(NGRP):
            pltpu.async_copy(
                t_hbm.at[cid].at[idx_v.at[p, pl.ds(j * GSZ, GSZ)]],
                rows_v.at[p, pl.ds(j * GSZ, GSZ)],
                sem_g[p],
            )

    def wait_gathers(p):
        pltpu.make_async_copy(
            t_hbm.at[0, pl.ds(0, LCH)], rows_v.at[0], sem_g[p]).wait()

    def products(p):
        def body(ci4, colv):
            base = ci4 * (4 * CL)
            r = [rows_v[p, base + l, :] for l in range(4 * CL)]
            accs = []
            for q in range(4):
                o = q * CL
                accs.append(
                    ((r[o] * r[o + 1]) * (r[o + 2] * r[o + 3])) *
                    ((r[o + 4] * r[o + 5]) * (r[o + 6] * r[o + 7])))
            for q in range(4):
                plsc.store_scatter(obuf, [iota16, colv + q], 1.0 - accs[q])
            return colv + 4
        lax.fori_loop(0, CCH // 4, body,
                      jnp.zeros((16,), jnp.int32), unroll=2)

    def write_out(chunk):
        ce = jnp.minimum(chunk, NCH - 1)
        pltpu.sync_copy(obuf, out_hbm.at[:, pl.ds(ce * CCH, CCH)])

    lit0 = jnp.minimum(c0, NCH - 1) * LCH
    pltpu.sync_copy(cidx_hbm.at[pl.ds(lit0, LCH)], idx_v.at[0])
    fire_gathers(0)
    stage_vn_async(1, c0 + 1)

    def pair_body(t, z):
        k = c0 + 2 * t
        wait_gathers(0)
        wait_vn(1)
        fire_gathers(1)
        stage_vn_async(0, k + 2)
        products(0)
        write_out(k)
        wait_gathers(1)
        wait_vn(0)
        fire_gathers(0)
        stage_vn_async(1, k + 3)
        products(1)
        write_out(k + 1)
        return z

    lax.fori_loop(0, KPW // 2 - 1, pair_body, 0)

    wait_gathers(0)
    wait_vn(1)
    fire_gathers(1)
    products(0)
    write_out(c0 + KPW - 2)
    wait_gathers(1)
    products(1)
    write_out(c0 + KPW - 1)


@jax.jit
def _run(s, cidx):
    mesh = plsc.VectorSubcoreMesh(core_axis_name="c", subcore_axis_name="s")
    sc = functools.partial(
        pl.kernel,
        mesh=mesh,
        out_type=(
            jax.ShapeDtypeStruct((B, NC), jnp.float32),
            jax.ShapeDtypeStruct((2, 2 * NV, 16), jnp.float32),
        ),
        scratch_types=[
            pltpu.VMEM((16, BCOLS), jnp.float32),
            pltpu.VMEM((2, 2, BCOLS, 16), jnp.float32),
            pltpu.VMEM((2, LCH), jnp.int32),
            pltpu.VMEM((2, LCH, 16), jnp.float32),
            pltpu.VMEM((16, CCH), jnp.float32),
            pltpu.SemaphoreType.DMA,
            pltpu.SemaphoreType.DMA,
            pltpu.SemaphoreType.DMA,
            pltpu.SemaphoreType.DMA,
        ],
        compiler_params=pltpu.CompilerParams(
            use_tc_tiling_on_sc=False, needs_layout_passes=False,
            disable_bounds_checks=True),
    )(_sc_body)
    return sc(s, cidx)


def kernel(input, emb_weight, var_idx, neg):
    w = jnp.take(emb_weight, input, axis=0, mode='clip')
    s = jax.nn.sigmoid(w)
    cidx = neg * NV + var_idx - 1
    out, _ = _run(s, cidx)
    return out


# speedup vs baseline: 1.3391x; 1.0405x over previous
import functools

import jax
import jax.numpy as jnp
from jax import lax
from jax.experimental import pallas as pl
from jax.experimental.pallas import tpu as pltpu
from jax.experimental.pallas import tpu_sc as plsc

NV = 50000
NC = 100000
CL = 8
B = 16

BCOLS = 784
BITER = 4
NW = 32
CCH = 160
LCH = CCH * CL
NGRP = 10
GSZ = LCH // NGRP
NCH = NC // CCH
KPW = 20
TG = CCH // 16


def _sc_body(s_hbm, cidx_hbm, out_hbm, t_hbm,
             wbuf, tbuf, idx_v, rows_v, obuf,
             sem_v0, sem_v1, sem_g0, sem_g1):
    cid = lax.axis_index("c")
    sid = lax.axis_index("s")
    w = sid * 2 + cid
    iota16 = lax.iota(jnp.int32, 16)
    sem_v = (sem_v0, sem_v1)
    sem_g = (sem_g0, sem_g1)

    def drain_build():
        pltpu.make_async_copy(
            tbuf.at[0, 0], t_hbm.at[0, pl.ds(0, BCOLS)], sem_g0).wait()
        pltpu.make_async_copy(
            tbuf.at[0, 0], t_hbm.at[0, pl.ds(0, BCOLS)], sem_g0).wait()

    for it in range(BITER):
        r = sid * BITER + it
        start = jnp.minimum(r * BCOLS, NV - BCOLS)
        pltpu.sync_copy(s_hbm.at[:, pl.ds(start, BCOLS)], wbuf)
        if it >= 2:
            drain_build()
        tb = tbuf.at[it % 2]

        def build_quad(u4, colv):
            u = u4 * 4
            xs = [plsc.load_gather(wbuf, [iota16, colv + q])
                  for q in range(4)]
            for q in range(4):
                tb[0, u + q, :] = 1.0 - xs[q]
                tb[1, u + q, :] = xs[q]
            return colv + 4

        lax.fori_loop(0, BCOLS // 4, build_quad,
                      jnp.zeros((16,), jnp.int32), unroll=2)
        pltpu.async_copy(
            tb.at[0], t_hbm.at[cid, pl.ds(start, BCOLS)], sem_g0)
        pltpu.async_copy(
            tb.at[1], t_hbm.at[cid, pl.ds(NV + start, BCOLS)], sem_g0)
    drain_build()
    drain_build()
    plsc.subcore_barrier()

    c0 = w * KPW

    def stage_vn_async(p, chunk):
        lit = jnp.minimum(chunk, NCH - 1) * LCH
        pltpu.async_copy(cidx_hbm.at[pl.ds(lit, LCH)], idx_v.at[p], sem_v[p])

    def wait_vn(p):
        pltpu.make_async_copy(
            cidx_hbm.at[pl.ds(0, LCH)], idx_v.at[0], sem_v[p]).wait()

    def fire_gathers(p):
        for j in range(NGRP):
            pltpu.async_copy(
                t_hbm.at[cid].at[idx_v.at[p, pl.ds(j * GSZ, GSZ)]],
                rows_v.at[p, pl.ds(j * GSZ, GSZ)],
                sem_g[p],
            )

    def wait_gathers(p):
        pltpu.make_async_copy(
            t_hbm.at[0, pl.ds(0, LCH)], rows_v.at[0], sem_g[p]).wait()

    def products(p):
        def body(ci4, colv):
            base = ci4 * (4 * CL)
            r = [rows_v[p, base + l, :] for l in range(4 * CL)]
            accs = []
            for q in range(4):
                o = q * CL
                accs.append(
                    ((r[o] * r[o + 1]) * (r[o + 2] * r[o + 3])) *
                    ((r[o + 4] * r[o + 5]) * (r[o + 6] * r[o + 7])))
            for q in range(4):
                plsc.store_scatter(obuf, [iota16, colv + q], 1.0 - accs[q])
            return colv + 4
        lax.fori_loop(0, CCH // 4, body,
                      jnp.zeros((16,), jnp.int32), unroll=2)

    def write_out(chunk):
        ce = jnp.minimum(chunk, NCH - 1)
        pltpu.sync_copy(obuf, out_hbm.at[:, pl.ds(ce * CCH, CCH)])

    lit0 = jnp.minimum(c0, NCH - 1) * LCH
    pltpu.sync_copy(cidx_hbm.at[pl.ds(lit0, LCH)], idx_v.at[0])
    fire_gathers(0)
    stage_vn_async(1, c0 + 1)

    def pair_body(t, z):
        k = c0 + 2 * t
        wait_gathers(0)
        wait_vn(1)
        fire_gathers(1)
        stage_vn_async(0, k + 2)
        products(0)
        write_out(k)
        wait_gathers(1)
        wait_vn(0)
        fire_gathers(0)
        stage_vn_async(1, k + 3)
        products(1)
        write_out(k + 1)
        return z

    lax.fori_loop(0, KPW // 2 - 1, pair_body, 0)

    wait_gathers(0)
    wait_vn(1)
    fire_gathers(1)
    products(0)
    write_out(c0 + KPW - 2)
    wait_gathers(1)
    products(1)
    write_out(c0 + KPW - 1)


@jax.jit
def _run(s, cidx):
    mesh = plsc.VectorSubcoreMesh(core_axis_name="c", subcore_axis_name="s")
    sc = functools.partial(
        pl.kernel,
        mesh=mesh,
        out_type=(
            jax.ShapeDtypeStruct((B, NC), jnp.float32),
            jax.ShapeDtypeStruct((2, 2 * NV, 16), jnp.float32),
        ),
        scratch_types=[
            pltpu.VMEM((16, BCOLS), jnp.float32),
            pltpu.VMEM((2, 2, BCOLS, 16), jnp.float32),
            pltpu.VMEM((2, LCH), jnp.int32),
            pltpu.VMEM((2, LCH, 16), jnp.float32),
            pltpu.VMEM((16, CCH), jnp.float32),
            pltpu.SemaphoreType.DMA,
            pltpu.SemaphoreType.DMA,
            pltpu.SemaphoreType.DMA,
            pltpu.SemaphoreType.DMA,
        ],
        compiler_params=pltpu.CompilerParams(
            use_tc_tiling_on_sc=False, needs_layout_passes=False,
            disable_bounds_checks=True),
    )(_sc_body)
    return sc(s, cidx)


def kernel(input, emb_weight, var_idx, neg):
    w = jnp.take(emb_weight, input, axis=0, mode='clip')
    s = jax.nn.sigmoid(w)
    cidx = neg * NV + var_idx - 1
    out, _ = _run(s, cidx)
    return out
